# baseline (speedup 1.0000x reference)
import functools

import jax
import jax.numpy as jnp
from jax import lax
from jax.experimental import pallas as pl
from jax.experimental.pallas import tpu as pltpu
from jax.experimental.pallas import tpu_sc as plsc

N = 10000
NPAD = 10112
E = 320000
SROW = 500
EROWS = E // SROW
NTILES = 32
BLK_EDGE = EROWS // NTILES
BLK_COL = EROWS // 16
ROWS_PER_SUB = NPAD // 16
ZROWS = 128

_mesh = plsc.VectorSubcoreMesh(core_axis_name="c", subcore_axis_name="s")


def _zero_rows(ref, nrows, width, dtype=jnp.float32):
    lanes = 32 if dtype == jnp.bfloat16 else 16
    def body(i, _):
        for k in range(width // lanes):
            ref[i, pl.ds(k * lanes, lanes)] = jnp.zeros((lanes,), dtype)
        return 0
    lax.fori_loop(0, nrows, body, 0)


def _zero_acc_slice(zwin, acc, s):
    base = s * ROWS_PER_SUB
    nfull = ROWS_PER_SUB // ZROWS
    rem = ROWS_PER_SUB - nfull * ZROWS
    for k in range(nfull):
        pltpu.sync_copy(zwin, acc.at[pl.ds(base + k * ZROWS, ZROWS)])
    if rem:
        pltpu.sync_copy(zwin.at[pl.ds(0, rem)],
                        acc.at[pl.ds(base + nfull * ZROWS, rem)])


def _copy_acc_out(acc, out_hbm, c, s):
    base = s * ROWS_PER_SUB
    pltpu.sync_copy(acc.at[pl.ds(base, ROWS_PER_SUB)],
                    out_hbm.at[c, pl.ds(base, ROWS_PER_SUB)])


def _agg_pipeline(table, src_v, dst_v, rows, acc, sg, ss, nblocks):
    NB = 4
    assert nblocks % NB == 0

    def wait_gather(buf, sem):
        pltpu.make_async_copy(table.at[src_v.at[0]], buf, sem).wait()

    def wait_scatter(buf, sem):
        pltpu.make_async_copy(buf, acc.at[dst_v.at[0]], sem).wait()

    for j in range(NB - 1):
        pltpu.async_copy(table.at[src_v.at[j]], rows[j], sg[j])

    def body(i, _):
        for j in range(NB):
            k = NB * i + j
            jn = (j + NB - 1) % NB
            wait_gather(rows[j], sg[j])
            pltpu.async_copy(rows[j], acc.at[dst_v.at[k]], ss[j], add=True)

            @pl.when(k >= 1)
            def _():
                wait_scatter(rows[jn], ss[jn])

            @pl.when(k + NB - 1 < nblocks)
            def _():
                pltpu.async_copy(table.at[src_v.at[k + NB - 1]],
                                 rows[jn], sg[jn])
        return 0
    lax.fori_loop(0, nblocks // NB, body, 0)
    wait_scatter(rows[(nblocks - 1) % NB], ss[(nblocks - 1) % NB])


@functools.partial(
    pl.kernel,
    mesh=_mesh,
    compiler_params=pltpu.CompilerParams(use_tc_tiling_on_sc=False),
    out_type=jax.ShapeDtypeStruct((2, NPAD, 16), jnp.float32),
    scratch_types=[
        pltpu.VMEM((BLK_EDGE, SROW), jnp.int32),
        pltpu.VMEM((SROW, 16), jnp.float32),
        pltpu.VMEM((ZROWS, 16), jnp.float32),
        pltpu.VMEM_SHARED((NPAD, 16), jnp.float32),
        pltpu.SemaphoreType.DMA,
    ],
)
def _deg_call(ei_hbm, out_hbm, dst_v, ones_v, zwin_v, acc, sdeg):
    c = lax.axis_index("c")
    s = lax.axis_index("s")
    wid = c * 16 + s

    def fill(i, _):
        ones_v[i, :] = jnp.ones((16,), jnp.float32)
        return 0
    lax.fori_loop(0, SROW, fill, 0)
    _zero_rows(zwin_v, ZROWS, 16)
    _zero_acc_slice(zwin_v, acc, s)
    plsc.subcore_barrier()

    pltpu.sync_copy(ei_hbm.at[1, pl.ds(wid * BLK_EDGE, BLK_EDGE)], dst_v)

    def body(j, _):
        pltpu.async_copy(ones_v, acc.at[dst_v.at[j]], sdeg, add=True)
        return 0
    lax.fori_loop(0, BLK_EDGE, body, 0)

    def drain(j, _):
        pltpu.make_async_copy(ones_v, acc.at[dst_v.at[0]], sdeg).wait()
        return 0
    lax.fori_loop(0, BLK_EDGE, drain, 0)

    plsc.subcore_barrier()
    _copy_acc_out(acc, out_hbm, c, s)


@functools.partial(
    pl.kernel,
    mesh=_mesh,
    compiler_params=pltpu.CompilerParams(use_tc_tiling_on_sc=False),
    out_type=jax.ShapeDtypeStruct((2, NPAD, 32), jnp.bfloat16),
    scratch_types=[
        pltpu.VMEM((BLK_EDGE, SROW), jnp.int32),
        pltpu.VMEM((BLK_EDGE, SROW), jnp.int32),
        pltpu.VMEM((4, SROW, 32), jnp.bfloat16),
        pltpu.VMEM_SHARED((NPAD, 32), jnp.bfloat16),
        pltpu.VMEM_SHARED((NPAD, 32), jnp.bfloat16),
        pltpu.SemaphoreType.DMA,
        pltpu.SemaphoreType.DMA,
        pltpu.SemaphoreType.DMA,
        pltpu.SemaphoreType.DMA,
        pltpu.SemaphoreType.DMA,
        pltpu.SemaphoreType.DMA,
        pltpu.SemaphoreType.DMA,
        pltpu.SemaphoreType.DMA,
    ],
)
def _agg64(hs_hbm, ei_hbm, out_hbm,
           src_v, dst_v, rows4, acc, table,
           sg0, sg1, sg2, sg3, ss0, ss1, ss2, ss3):
    c = lax.axis_index("c")
    s = lax.axis_index("s")

    pltpu.sync_copy(hs_hbm.at[c, pl.ds(s * ROWS_PER_SUB, ROWS_PER_SUB)],
                    table.at[pl.ds(s * ROWS_PER_SUB, ROWS_PER_SUB)])
    _zero_rows(rows4.at[0, pl.ds(0, ZROWS)], ZROWS, 32, jnp.bfloat16)
    _zero_acc_slice(rows4.at[0, pl.ds(0, ZROWS)], acc, s)
    plsc.subcore_barrier()

    for phase in range(2):
        base = s * BLK_COL + phase * BLK_EDGE
        pltpu.sync_copy(ei_hbm.at[0, pl.ds(base, BLK_EDGE)], src_v)
        pltpu.sync_copy(ei_hbm.at[1, pl.ds(base, BLK_EDGE)], dst_v)
        _agg_pipeline(table, src_v, dst_v,
                      [rows4.at[0], rows4.at[1], rows4.at[2], rows4.at[3]],
                      acc, [sg0, sg1, sg2, sg3], [ss0, ss1, ss2, ss3],
                      BLK_EDGE)

    plsc.subcore_barrier()
    _copy_acc_out(acc, out_hbm, c, s)


@functools.partial(
    pl.kernel,
    mesh=_mesh,
    compiler_params=pltpu.CompilerParams(use_tc_tiling_on_sc=False),
    out_type=jax.ShapeDtypeStruct((2, NPAD, 16), jnp.float32),
    scratch_types=[
        pltpu.VMEM((BLK_EDGE, SROW), jnp.int32),
        pltpu.VMEM((BLK_EDGE, SROW), jnp.int32),
        pltpu.VMEM((4, SROW, 16), jnp.float32),
        pltpu.VMEM_SHARED((NPAD, 16), jnp.float32),
        pltpu.VMEM_SHARED((NPAD, 16), jnp.float32),
        pltpu.SemaphoreType.DMA,
        pltpu.SemaphoreType.DMA,
        pltpu.SemaphoreType.DMA,
        pltpu.SemaphoreType.DMA,
        pltpu.SemaphoreType.DMA,
        pltpu.SemaphoreType.DMA,
        pltpu.SemaphoreType.DMA,
        pltpu.SemaphoreType.DMA,
    ],
)
def _agg16(hs_hbm, ei_hbm, out_hbm,
           src_v, dst_v, rows4, acc, table,
           sg0, sg1, sg2, sg3, ss0, ss1, ss2, ss3):
    c = lax.axis_index("c")
    s = lax.axis_index("s")
    wid = c * 16 + s

    pltpu.sync_copy(hs_hbm.at[pl.ds(s * ROWS_PER_SUB, ROWS_PER_SUB)],
                    table.at[pl.ds(s * ROWS_PER_SUB, ROWS_PER_SUB)])
    _zero_rows(rows4.at[0, pl.ds(0, ZROWS)], ZROWS, 16)
    _zero_acc_slice(rows4.at[0, pl.ds(0, ZROWS)], acc, s)
    plsc.subcore_barrier()

    pltpu.sync_copy(ei_hbm.at[0, pl.ds(wid * BLK_EDGE, BLK_EDGE)], src_v)
    pltpu.sync_copy(ei_hbm.at[1, pl.ds(wid * BLK_EDGE, BLK_EDGE)], dst_v)

    _agg_pipeline(table, src_v, dst_v,
                  [rows4.at[0], rows4.at[1], rows4.at[2], rows4.at[3]], acc,
                  [sg0, sg1, sg2, sg3], [ss0, ss1, ss2, ss3], BLK_EDGE)

    plsc.subcore_barrier()
    _copy_acc_out(acc, out_hbm, c, s)


RB1 = NPAD // 8
RB3 = N // 10


def _tc0_body(x_ref, w1_ref, h1_ref):
    h1_ref[pl.ds(0, N)] = jnp.dot(x_ref[...], w1_ref[...],
                                  preferred_element_type=jnp.float32)
    h1_ref[pl.ds(N, NPAD - N)] = jnp.zeros((NPAD - N, 64), jnp.float32)


def _tc1_body(degp_ref, h1_ref, hs1_ref, dis_ref):
    deg = degp_ref[0, :, 0:1] + degp_ref[1, :, 0:1] + 1.0
    dis = lax.rsqrt(deg)
    hs1 = (h1_ref[...] * dis).astype(jnp.bfloat16)
    hs1_ref[0] = hs1[:, :32]
    hs1_ref[1] = hs1[:, 32:]
    dis_ref[...] = dis


def _tc2_body(p_ref, hs1_ref, dis_ref, w2_ref, b1_ref, hs2_ref):
    dis = dis_ref[...]
    agg = jnp.concatenate(
        [p_ref[0].astype(jnp.float32) + hs1_ref[0].astype(jnp.float32),
         p_ref[1].astype(jnp.float32) + hs1_ref[1].astype(jnp.float32)],
        axis=1)
    out1 = dis * agg + b1_ref[...]
    r = jnp.maximum(out1, 0.0)
    h2 = jnp.dot(r, w2_ref[...], preferred_element_type=jnp.float32)
    hs2_ref[...] = h2 * dis


def _tc3_body(q_ref, hs2_ref, dis_ref, b2_ref, out_ref):
    dis = dis_ref[...]
    agg = q_ref[0] + q_ref[1] + hs2_ref[...]
    out_ref[...] = dis * agg + b2_ref[...]


_tc0 = pl.pallas_call(
    _tc0_body,
    out_shape=jax.ShapeDtypeStruct((NPAD, 64), jnp.float32),
)

_tc1 = pl.pallas_call(
    _tc1_body,
    grid=(8,),
    in_specs=[pl.BlockSpec((2, RB1, 16), lambda i: (0, i, 0)),
              pl.BlockSpec((RB1, 64), lambda i: (i, 0))],
    out_specs=[pl.BlockSpec((2, RB1, 32), lambda i: (0, i, 0)),
               pl.BlockSpec((RB1, 1), lambda i: (i, 0))],
    out_shape=[jax.ShapeDtypeStruct((2, NPAD, 32), jnp.bfloat16),
               jax.ShapeDtypeStruct((NPAD, 1), jnp.float32)],
)

_tc2 = pl.pallas_call(
    _tc2_body,
    grid=(8,),
    in_specs=[pl.BlockSpec((2, RB1, 32), lambda i: (0, i, 0)),
              pl.BlockSpec((2, RB1, 32), lambda i: (0, i, 0)),
              pl.BlockSpec((RB1, 1), lambda i: (i, 0)),
              pl.BlockSpec((64, 16), lambda i: (0, 0)),
              pl.BlockSpec((1, 64), lambda i: (0, 0))],
    out_specs=pl.BlockSpec((RB1, 16), lambda i: (i, 0)),
    out_shape=jax.ShapeDtypeStruct((NPAD, 16), jnp.float32),
)

_tc3 = pl.pallas_call(
    _tc3_body,
    grid=(10,),
    in_specs=[pl.BlockSpec((2, RB3, 16), lambda i: (0, i, 0)),
              pl.BlockSpec((RB3, 16), lambda i: (i, 0)),
              pl.BlockSpec((RB3, 1), lambda i: (i, 0)),
              pl.BlockSpec((1, 16), lambda i: (0, 0))],
    out_specs=pl.BlockSpec((RB3, 16), lambda i: (i, 0)),
    out_shape=jax.ShapeDtypeStruct((N, 16), jnp.float32),
)


def kernel(x, edge_index, W1, b1, W2, b2):
    ei3 = edge_index.reshape(2, EROWS, SROW)

    h1 = _tc0(x, W1)
    degp = _deg_call(ei3)
    hs1s, dis = _tc1(degp, h1)
    p = _agg64(hs1s, ei3)
    hs2 = _tc2(p, hs1s, dis, W2, b1.reshape(1, 64))
    q = _agg16(hs2, ei3)
    return _tc3(q, hs2, dis, b2.reshape(1, 16))


# speedup vs baseline: 1.0077x; 1.0077x over previous
import functools

import jax
import jax.numpy as jnp
from jax import lax
from jax.experimental import pallas as pl
from jax.experimental.pallas import tpu as pltpu
from jax.experimental.pallas import tpu_sc as plsc

N = 10000
NPAD = 10112
E = 320000
SROW = 500
EROWS = E // SROW
NTILES = 32
BLK_EDGE = EROWS // NTILES
BLK_COL = EROWS // 16
ROWS_PER_SUB = NPAD // 16
ZROWS = 128

_mesh = plsc.VectorSubcoreMesh(core_axis_name="c", subcore_axis_name="s")


def _zero_rows(ref, nrows, width, dtype=jnp.float32):
    lanes = 32 if dtype == jnp.bfloat16 else 16
    def body(i, _):
        for k in range(width // lanes):
            ref[i, pl.ds(k * lanes, lanes)] = jnp.zeros((lanes,), dtype)
        return 0
    lax.fori_loop(0, nrows, body, 0)


def _zero_acc_slice(zwin, acc, s):
    base = s * ROWS_PER_SUB
    nfull = ROWS_PER_SUB // ZROWS
    rem = ROWS_PER_SUB - nfull * ZROWS
    for k in range(nfull):
        pltpu.sync_copy(zwin, acc.at[pl.ds(base + k * ZROWS, ZROWS)])
    if rem:
        pltpu.sync_copy(zwin.at[pl.ds(0, rem)],
                        acc.at[pl.ds(base + nfull * ZROWS, rem)])


def _copy_acc_out(acc, out_hbm, c, s):
    base = s * ROWS_PER_SUB
    pltpu.sync_copy(acc.at[pl.ds(base, ROWS_PER_SUB)],
                    out_hbm.at[c, pl.ds(base, ROWS_PER_SUB)])


def _agg_pipeline(table, src_v, dst_v, rows, acc, sg, ss, nblocks):
    NB = 4
    assert nblocks % NB == 0

    def wait_gather(buf, sem):
        pltpu.make_async_copy(table.at[src_v.at[0]], buf, sem).wait()

    def wait_scatter(buf, sem):
        pltpu.make_async_copy(buf, acc.at[dst_v.at[0]], sem).wait()

    for j in range(NB - 1):
        pltpu.async_copy(table.at[src_v.at[j]], rows[j], sg[j])

    def body(i, _):
        for j in range(NB):
            k = NB * i + j
            jn = (j + NB - 1) % NB
            wait_gather(rows[j], sg[j])
            pltpu.async_copy(rows[j], acc.at[dst_v.at[k]], ss[j], add=True)

            @pl.when(k >= 1)
            def _():
                wait_scatter(rows[jn], ss[jn])

            @pl.when(k + NB - 1 < nblocks)
            def _():
                pltpu.async_copy(table.at[src_v.at[k + NB - 1]],
                                 rows[jn], sg[jn])
        return 0
    lax.fori_loop(0, nblocks // NB, body, 0)
    wait_scatter(rows[(nblocks - 1) % NB], ss[(nblocks - 1) % NB])


@functools.partial(
    pl.kernel,
    mesh=_mesh,
    compiler_params=pltpu.CompilerParams(use_tc_tiling_on_sc=False),
    out_type=jax.ShapeDtypeStruct((2, NPAD, 16), jnp.float32),
    scratch_types=[
        pltpu.VMEM((BLK_EDGE, SROW), jnp.int32),
        pltpu.VMEM((SROW, 16), jnp.float32),
        pltpu.VMEM((ZROWS, 16), jnp.float32),
        pltpu.VMEM_SHARED((NPAD, 16), jnp.float32),
        pltpu.SemaphoreType.DMA,
    ],
)
def _deg_call(ei_hbm, out_hbm, dst_v, ones_v, zwin_v, acc, sdeg):
    c = lax.axis_index("c")
    s = lax.axis_index("s")
    wid = c * 16 + s

    def fill(i, _):
        ones_v[i, :] = jnp.ones((16,), jnp.float32)
        return 0
    lax.fori_loop(0, SROW, fill, 0)
    _zero_rows(zwin_v, ZROWS, 16)
    _zero_acc_slice(zwin_v, acc, s)
    plsc.subcore_barrier()

    pltpu.sync_copy(ei_hbm.at[1, pl.ds(wid * BLK_EDGE, BLK_EDGE)], dst_v)

    def body(j, _):
        pltpu.async_copy(ones_v, acc.at[dst_v.at[j]], sdeg, add=True)
        return 0
    lax.fori_loop(0, BLK_EDGE, body, 0)

    def drain(j, _):
        pltpu.make_async_copy(ones_v, acc.at[dst_v.at[0]], sdeg).wait()
        return 0
    lax.fori_loop(0, BLK_EDGE, drain, 0)

    plsc.subcore_barrier()
    _copy_acc_out(acc, out_hbm, c, s)


@functools.partial(
    pl.kernel,
    mesh=_mesh,
    compiler_params=pltpu.CompilerParams(use_tc_tiling_on_sc=False),
    out_type=jax.ShapeDtypeStruct((2, NPAD, 32), jnp.bfloat16),
    scratch_types=[
        pltpu.VMEM((BLK_EDGE, SROW), jnp.int32),
        pltpu.VMEM((BLK_EDGE, SROW), jnp.int32),
        pltpu.VMEM((4, SROW, 32), jnp.bfloat16),
        pltpu.VMEM_SHARED((NPAD, 32), jnp.bfloat16),
        pltpu.VMEM_SHARED((NPAD, 32), jnp.bfloat16),
        pltpu.SemaphoreType.DMA,
        pltpu.SemaphoreType.DMA,
        pltpu.SemaphoreType.DMA,
        pltpu.SemaphoreType.DMA,
        pltpu.SemaphoreType.DMA,
        pltpu.SemaphoreType.DMA,
        pltpu.SemaphoreType.DMA,
        pltpu.SemaphoreType.DMA,
    ],
)
def _agg64(hs_hbm, ei_hbm, out_hbm,
           src_v, dst_v, rows4, acc, table,
           sg0, sg1, sg2, sg3, ss0, ss1, ss2, ss3):
    c = lax.axis_index("c")
    s = lax.axis_index("s")

    pltpu.sync_copy(hs_hbm.at[c, pl.ds(s * ROWS_PER_SUB, ROWS_PER_SUB)],
                    table.at[pl.ds(s * ROWS_PER_SUB, ROWS_PER_SUB)])
    _zero_rows(rows4.at[0, pl.ds(0, ZROWS)], ZROWS, 32, jnp.bfloat16)
    _zero_acc_slice(rows4.at[0, pl.ds(0, ZROWS)], acc, s)
    plsc.subcore_barrier()

    for phase in range(2):
        base = s * BLK_COL + phase * BLK_EDGE
        pltpu.sync_copy(ei_hbm.at[0, pl.ds(base, BLK_EDGE)], src_v)
        pltpu.sync_copy(ei_hbm.at[1, pl.ds(base, BLK_EDGE)], dst_v)
        _agg_pipeline(table, src_v, dst_v,
                      [rows4.at[0], rows4.at[1], rows4.at[2], rows4.at[3]],
                      acc, [sg0, sg1, sg2, sg3], [ss0, ss1, ss2, ss3],
                      BLK_EDGE)

    plsc.subcore_barrier()
    _copy_acc_out(acc, out_hbm, c, s)


@functools.partial(
    pl.kernel,
    mesh=_mesh,
    compiler_params=pltpu.CompilerParams(use_tc_tiling_on_sc=False),
    out_type=jax.ShapeDtypeStruct((2, NPAD, 16), jnp.float32),
    scratch_types=[
        pltpu.VMEM((BLK_EDGE, SROW), jnp.int32),
        pltpu.VMEM((BLK_EDGE, SROW), jnp.int32),
        pltpu.VMEM((4, SROW, 16), jnp.float32),
        pltpu.VMEM_SHARED((NPAD, 16), jnp.float32),
        pltpu.VMEM_SHARED((NPAD, 16), jnp.float32),
        pltpu.SemaphoreType.DMA,
        pltpu.SemaphoreType.DMA,
        pltpu.SemaphoreType.DMA,
        pltpu.SemaphoreType.DMA,
        pltpu.SemaphoreType.DMA,
        pltpu.SemaphoreType.DMA,
        pltpu.SemaphoreType.DMA,
        pltpu.SemaphoreType.DMA,
    ],
)
def _agg16(hs_hbm, ei_hbm, out_hbm,
           src_v, dst_v, rows4, acc, table,
           sg0, sg1, sg2, sg3, ss0, ss1, ss2, ss3):
    c = lax.axis_index("c")
    s = lax.axis_index("s")
    wid = c * 16 + s

    pltpu.sync_copy(hs_hbm.at[pl.ds(s * ROWS_PER_SUB, ROWS_PER_SUB)],
                    table.at[pl.ds(s * ROWS_PER_SUB, ROWS_PER_SUB)])
    _zero_rows(rows4.at[0, pl.ds(0, ZROWS)], ZROWS, 16)
    _zero_acc_slice(rows4.at[0, pl.ds(0, ZROWS)], acc, s)
    plsc.subcore_barrier()

    pltpu.sync_copy(ei_hbm.at[0, pl.ds(wid * BLK_EDGE, BLK_EDGE)], src_v)
    pltpu.sync_copy(ei_hbm.at[1, pl.ds(wid * BLK_EDGE, BLK_EDGE)], dst_v)

    _agg_pipeline(table, src_v, dst_v,
                  [rows4.at[0], rows4.at[1], rows4.at[2], rows4.at[3]], acc,
                  [sg0, sg1, sg2, sg3], [ss0, ss1, ss2, ss3], BLK_EDGE)

    plsc.subcore_barrier()
    _copy_acc_out(acc, out_hbm, c, s)


def _tc0_body(x_ref, w1_ref, h1_ref):
    h1_ref[...] = jnp.dot(x_ref[...], w1_ref[...],
                          preferred_element_type=jnp.float32)


def _tc1_body(degp_ref, h1_ref, hs1_ref, dis_ref):
    deg = degp_ref[0, :, 0:1] + degp_ref[1, :, 0:1] + 1.0
    dis = lax.rsqrt(deg)
    hs1 = (h1_ref[...] * dis[:N]).astype(jnp.bfloat16)
    hs1_ref[0, pl.ds(0, N)] = hs1[:, :32]
    hs1_ref[1, pl.ds(0, N)] = hs1[:, 32:]
    zt = jnp.zeros((NPAD - N, 32), jnp.bfloat16)
    hs1_ref[0, pl.ds(N, NPAD - N)] = zt
    hs1_ref[1, pl.ds(N, NPAD - N)] = zt
    dis_ref[...] = dis


def _tc2_body(p_ref, hs1_ref, dis_ref, w2_ref, b1_ref, hs2_ref):
    dis = dis_ref[...]
    agg = jnp.concatenate(
        [p_ref[0].astype(jnp.float32) + hs1_ref[0].astype(jnp.float32),
         p_ref[1].astype(jnp.float32) + hs1_ref[1].astype(jnp.float32)],
        axis=1)
    out1 = dis * agg + b1_ref[...]
    r = jnp.maximum(out1, 0.0)
    h2 = jnp.dot(r, w2_ref[...], preferred_element_type=jnp.float32)
    hs2_ref[...] = h2 * dis


def _tc3_body(q_ref, hs2_ref, dis_ref, b2_ref, out_ref):
    dis = dis_ref[...]
    agg = q_ref[0] + q_ref[1] + hs2_ref[...]
    out_ref[...] = dis * agg + b2_ref[...]


_tc0 = pl.pallas_call(
    _tc0_body,
    out_shape=jax.ShapeDtypeStruct((N, 64), jnp.float32),
)

_tc1 = pl.pallas_call(
    _tc1_body,
    out_shape=[jax.ShapeDtypeStruct((2, NPAD, 32), jnp.bfloat16),
               jax.ShapeDtypeStruct((NPAD, 1), jnp.float32)],
)

_tc2 = pl.pallas_call(
    _tc2_body,
    out_shape=jax.ShapeDtypeStruct((NPAD, 16), jnp.float32),
)

_tc3 = pl.pallas_call(
    _tc3_body,
    out_shape=jax.ShapeDtypeStruct((NPAD, 16), jnp.float32),
)


def kernel(x, edge_index, W1, b1, W2, b2):
    ei3 = edge_index.reshape(2, EROWS, SROW)

    h1 = _tc0(x, W1)
    degp = _deg_call(ei3)
    hs1s, dis = _tc1(degp, h1)
    p = _agg64(hs1s, ei3)
    hs2 = _tc2(p, hs1s, dis, W2, b1.reshape(1, 64))
    q = _agg16(hs2, ei3)
    out = _tc3(q, hs2, dis, b2.reshape(1, 16))
    return out[:N]
